# baseline (speedup 1.0000x reference)
import functools

import jax
import jax.numpy as jnp
from jax import lax
from jax.experimental import pallas as pl
from jax.experimental.pallas import tpu as pltpu
from jax.experimental.pallas import tpu_sc as plsc

F32 = jnp.float32

NC = 2
NS = 16
NW = NC * NS
CH = 256

_MESH = plsc.VectorSubcoreMesh(core_axis_name="c", subcore_axis_name="s")


def _sc_deg(n_pad, n_chunks):
  def body(dst, zeros8, ones_h, degp, dstv, onesv, zbuf, dacc):
    c = lax.axis_index("c")
    s = lax.axis_index("s")
    wid = s * NC + c
    rpt = n_pad // NS
    base = s * rpt
    pltpu.sync_copy(zeros8, zbuf)
    off = 0
    while off < rpt:
      w = min(CH, rpt - off)
      pltpu.sync_copy(zbuf.at[pl.ds(0, w)], dacc.at[pl.ds(base + off, w)])
      off += w
    pltpu.sync_copy(ones_h, onesv)
    pltpu.sync_copy(dst.at[wid], dstv)
    plsc.subcore_barrier()

    def chunk(i, carry):
      pltpu.sync_copy(onesv, dacc.at[dstv.at[i]], add=True)
      return carry

    lax.fori_loop(0, n_chunks, chunk, 0)
    plsc.subcore_barrier()
    off = 0
    while off < rpt:
      w = min(CH, rpt - off)
      pltpu.sync_copy(dacc.at[pl.ds(base + off, w)], zbuf.at[pl.ds(0, w)])
      pltpu.sync_copy(zbuf.at[pl.ds(0, w)], degp.at[c, pl.ds(base + off, w)])
      off += w

  return pl.kernel(
      body,
      out_type=[jax.ShapeDtypeStruct((NC, n_pad, 8), F32)],
      mesh=_MESH,
      compiler_params=pltpu.CompilerParams(use_tc_tiling_on_sc=False),
      scratch_types=[
          pltpu.VMEM((n_chunks, CH), jnp.int32),
          pltpu.VMEM((CH, 8), F32),
          pltpu.VMEM((CH, 8), F32),
          pltpu.VMEM_SHARED((n_pad, 8), F32),
      ],
  )


def _sc_agg64(n_pad, n_chunks):
  assert n_chunks % 2 == 0

  def body(table2, src, dst, zeros64, part,
           srcv, dstv, rows0, rows1, acc, gs0, gs1, ss0, ss1):
    c = lax.axis_index("c")
    s = lax.axis_index("s")
    rpt = n_pad // NS
    base = s * rpt
    pltpu.sync_copy(zeros64, rows0)
    off = 0
    while off < rpt:
      w = min(CH, rpt - off)
      pltpu.sync_copy(rows0.at[pl.ds(0, w)], acc.at[pl.ds(base + off, w)])
      off += w
    pltpu.sync_copy(src.at[s], srcv)
    pltpu.sync_copy(dst.at[s], dstv)
    plsc.subcore_barrier()

    tab = table2.at[c]

    def pair(j, carry):
      for k, (rows, gs, ss) in ((0, (rows0, gs0, ss0)),
                                (1, (rows1, gs1, ss1))):
        i = 2 * j + k

        @pl.when(j > 0)
        def _():
          pltpu.make_async_copy(rows, acc.at[dstv.at[0]], ss).wait()

        pltpu.async_copy(tab.at[srcv.at[i]], rows, gs).wait()
        pltpu.async_copy(rows, acc.at[dstv.at[i]], ss, add=True)
      return carry

    lax.fori_loop(0, n_chunks // 2, pair, 0)
    pltpu.make_async_copy(rows0, acc.at[dstv.at[0]], ss0).wait()
    pltpu.make_async_copy(rows1, acc.at[dstv.at[0]], ss1).wait()
    plsc.subcore_barrier()
    off = 0
    while off < rpt:
      w = min(CH, rpt - off)
      pltpu.sync_copy(acc.at[pl.ds(base + off, w)], rows0.at[pl.ds(0, w)])
      pltpu.sync_copy(rows0.at[pl.ds(0, w)], part.at[c, pl.ds(base + off, w)])
      off += w

  return pl.kernel(
      body,
      out_type=[jax.ShapeDtypeStruct((NC, n_pad, 64), F32)],
      mesh=_MESH,
      compiler_params=pltpu.CompilerParams(use_tc_tiling_on_sc=False),
      scratch_types=[
          pltpu.VMEM((n_chunks, CH), jnp.int32),
          pltpu.VMEM((n_chunks, CH), jnp.int32),
          pltpu.VMEM((CH, 64), F32),
          pltpu.VMEM((CH, 64), F32),
          pltpu.VMEM_SHARED((n_pad, 64), F32),
          pltpu.SemaphoreType.DMA,
          pltpu.SemaphoreType.DMA,
          pltpu.SemaphoreType.DMA,
          pltpu.SemaphoreType.DMA,
      ],
  )


def _sc_agg8(n_pad, n_chunks):
  assert n_chunks % 2 == 0

  def body(table, src, dst, zeros8, part,
           srcv, dstv, rows0, rows1, acc, gs0, gs1, ss0, ss1):
    c = lax.axis_index("c")
    s = lax.axis_index("s")
    wid = s * NC + c
    rpt = n_pad // NS
    base = s * rpt
    pltpu.sync_copy(zeros8, rows0)
    off = 0
    while off < rpt:
      w = min(CH, rpt - off)
      pltpu.sync_copy(rows0.at[pl.ds(0, w)], acc.at[pl.ds(base + off, w)])
      off += w
    pltpu.sync_copy(src.at[wid], srcv)
    pltpu.sync_copy(dst.at[wid], dstv)
    plsc.subcore_barrier()

    def pair(j, carry):
      for k, (rows, gs, ss) in ((0, (rows0, gs0, ss0)),
                                (1, (rows1, gs1, ss1))):
        i = 2 * j + k

        @pl.when(j > 0)
        def _():
          pltpu.make_async_copy(rows, acc.at[dstv.at[0]], ss).wait()

        pltpu.async_copy(table.at[srcv.at[i]], rows, gs).wait()
        pltpu.async_copy(rows, acc.at[dstv.at[i]], ss, add=True)
      return carry

    lax.fori_loop(0, n_chunks // 2, pair, 0)
    pltpu.make_async_copy(rows0, acc.at[dstv.at[0]], ss0).wait()
    pltpu.make_async_copy(rows1, acc.at[dstv.at[0]], ss1).wait()
    plsc.subcore_barrier()
    off = 0
    while off < rpt:
      w = min(CH, rpt - off)
      pltpu.sync_copy(acc.at[pl.ds(base + off, w)], rows0.at[pl.ds(0, w)])
      pltpu.sync_copy(rows0.at[pl.ds(0, w)], part.at[c, pl.ds(base + off, w)])
      off += w

  return pl.kernel(
      body,
      out_type=[jax.ShapeDtypeStruct((NC, n_pad, 8), F32)],
      mesh=_MESH,
      compiler_params=pltpu.CompilerParams(use_tc_tiling_on_sc=False),
      scratch_types=[
          pltpu.VMEM((n_chunks, CH), jnp.int32),
          pltpu.VMEM((n_chunks, CH), jnp.int32),
          pltpu.VMEM((CH, 8), F32),
          pltpu.VMEM((CH, 8), F32),
          pltpu.VMEM_SHARED((n_pad, 8), F32),
          pltpu.SemaphoreType.DMA,
          pltpu.SemaphoreType.DMA,
          pltpu.SemaphoreType.DMA,
          pltpu.SemaphoreType.DMA,
      ],
  )


def _tc_transform(n, d_in, bm):
  def body(h, wl, wr, b, ztab, sout):
    hb = h[...]
    z = jnp.dot(hb, wl[...], preferred_element_type=F32)
    ztab[0] = z[:, :64]
    ztab[1] = z[:, 64:]
    sout[...] = jnp.dot(hb, wr[...], preferred_element_type=F32) + b[...]

  return pl.pallas_call(
      body,
      grid=(n // bm,),
      in_specs=[
          pl.BlockSpec((bm, d_in), lambda i: (i, 0)),
          pl.BlockSpec((d_in, 128), lambda i: (0, 0)),
          pl.BlockSpec((d_in, 128), lambda i: (0, 0)),
          pl.BlockSpec((1, 128), lambda i: (0, 0)),
      ],
      out_specs=[
          pl.BlockSpec((2, bm, 64), lambda i: (0, i, 0)),
          pl.BlockSpec((bm, 128), lambda i: (i, 0)),
      ],
      out_shape=[
          jax.ShapeDtypeStruct((2, n, 64), F32),
          jax.ShapeDtypeStruct((n, 128), F32),
      ],
  )


def _tc_combine_transform(n, bm, table_out, d_extra):
  def body(p, degp, s, wc, bc, *outs):
    deg = degp[0, :, 0:1] + degp[1, :, 0:1]
    invd = 1.0 / jnp.maximum(deg, 1.0)
    agg = jnp.concatenate([p[0], p[1]], axis=1)
    h = jnp.maximum(agg * invd + s[...], 0.0)
    y = jnp.dot(h, wc[...], preferred_element_type=F32) + bc[...]
    if table_out:
      outs[0][0] = y[:, :64]
      outs[0][1] = y[:, 64:128]
      outs[1][...] = y[:, 128:]
    else:
      outs[0][...] = y

  d_out = (128 if table_out else 0) + d_extra
  if table_out:
    out_specs = [pl.BlockSpec((2, bm, 64), lambda i: (0, i, 0)),
                 pl.BlockSpec((bm, d_extra), lambda i: (i, 0))]
    out_shape = [jax.ShapeDtypeStruct((2, n, 64), F32),
                 jax.ShapeDtypeStruct((n, d_extra), F32)]
  else:
    out_specs = [pl.BlockSpec((bm, d_extra), lambda i: (i, 0))]
    out_shape = [jax.ShapeDtypeStruct((n, d_extra), F32)]

  return pl.pallas_call(
      body,
      grid=(n // bm,),
      in_specs=[
          pl.BlockSpec((2, bm, 64), lambda i: (0, i, 0)),
          pl.BlockSpec((2, bm, 8), lambda i: (0, i, 0)),
          pl.BlockSpec((bm, 128), lambda i: (i, 0)),
          pl.BlockSpec((128, d_out), lambda i: (0, 0)),
          pl.BlockSpec((1, d_out), lambda i: (0, 0)),
      ],
      out_specs=out_specs,
      out_shape=out_shape,
  )


def _tc_final(n, bm):
  def body(p, degp, t3, out):
    deg = degp[0, :, 0:1] + degp[1, :, 0:1]
    invd = 1.0 / jnp.maximum(deg, 1.0)
    agg = p[0, :, 0:1] + p[1, :, 0:1]
    out[...] = agg * invd + t3[:, 1:2]

  return pl.pallas_call(
      body,
      grid=(n // bm,),
      in_specs=[
          pl.BlockSpec((2, bm, 8), lambda i: (0, i, 0)),
          pl.BlockSpec((2, bm, 8), lambda i: (0, i, 0)),
          pl.BlockSpec((bm, 8), lambda i: (i, 0)),
      ],
      out_specs=pl.BlockSpec((bm, 1), lambda i: (i, 0)),
      out_shape=jax.ShapeDtypeStruct((n, 1), F32),
  )


def _pad_edges(idx, nsplit, fill):
  per = idx.shape[0] // nsplit
  n_chunks = 2 * pl.cdiv(per, 2 * CH)
  pad = n_chunks * CH - per
  out = jnp.pad(idx.reshape(nsplit, per), ((0, 0), (0, pad)),
                constant_values=fill)
  return out.reshape(nsplit, n_chunks, CH), n_chunks


def kernel(x, edge_index, W1l, b1l, W1r, W2l, b2l, W2r, W3l, b3l, W3r):
  n, d_in = x.shape
  e = edge_index.shape[1]
  assert e % NW == 0 and e % NS == 0
  n_pad = pl.cdiv(n, 128) * 128
  bm = 2000

  src32, ch32 = _pad_edges(edge_index[0], NW, 0)
  dst32, _ = _pad_edges(edge_index[1], NW, n)
  src16, ch16 = _pad_edges(edge_index[0], NS, 0)
  dst16, _ = _pad_edges(edge_index[1], NS, n)
  zeros64 = jnp.zeros((CH, 64), F32)
  zeros8 = jnp.zeros((CH, 8), F32)
  ones_h = jnp.ones((CH, 8), F32)

  deg_k = _sc_deg(n_pad, ch32)
  agg64 = _sc_agg64(n_pad, ch16)
  agg8 = _sc_agg8(n_pad, ch32)

  (degp,) = deg_k(dst32, zeros8, ones_h)

  z1, s1 = _tc_transform(n, d_in, bm)(x, W1l.T, W1r.T, b1l[None, :])
  (p1,) = agg64(z1, src16, dst16, zeros64)

  wc2 = jnp.concatenate([W2l.T, W2r.T], axis=1)
  bc2 = jnp.concatenate([jnp.zeros((128,), F32), b2l])[None, :]
  z2, s2 = _tc_combine_transform(n, bm, True, 128)(p1, degp, s1, wc2, bc2)
  (p2,) = agg64(z2, src16, dst16, zeros64)

  wc3 = jnp.concatenate([W3l.T, W3r.T, jnp.zeros((128, 6), F32)], axis=1)
  bc3 = jnp.concatenate([jnp.zeros((1,), F32), b3l,
                         jnp.zeros((6,), F32)])[None, :]
  (t3,) = _tc_combine_transform(n, bm, False, 8)(p2, degp, s2, wc3, bc3)
  (p3,) = agg8(t3, src32, dst32, zeros8)

  return _tc_final(n, bm)(p3, degp, t3)


# speedup vs baseline: 1.0267x; 1.0267x over previous
import functools

import jax
import jax.numpy as jnp
from jax import lax
from jax.experimental import pallas as pl
from jax.experimental.pallas import tpu as pltpu
from jax.experimental.pallas import tpu_sc as plsc

F32 = jnp.float32

NC = 2
NS = 16
NW = NC * NS
CH = 128

_MESH = plsc.VectorSubcoreMesh(core_axis_name="c", subcore_axis_name="s")


def _sc_deg(n_pad, n_chunks):
  def body(dst, zeros8, ones_h, degp, dstv, onesv, zbuf, dacc, ds0, ds1):
    c = lax.axis_index("c")
    s = lax.axis_index("s")
    wid = s * NC + c
    rpt = n_pad // NS
    base = s * rpt
    pltpu.sync_copy(zeros8, zbuf)
    off = 0
    while off < rpt:
      w = min(CH, rpt - off)
      pltpu.sync_copy(zbuf.at[pl.ds(0, w)], dacc.at[pl.ds(base + off, w)])
      off += w
    pltpu.sync_copy(ones_h, onesv)
    pltpu.sync_copy(dst.at[wid], dstv)
    plsc.subcore_barrier()

    def pairs(j, carry):
      for k, ds in ((0, ds0), (1, ds1)):
        i = 2 * j + k

        @pl.when(j > 0)
        def _():
          pltpu.make_async_copy(onesv, dacc.at[dstv.at[0]], ds).wait()

        pltpu.async_copy(onesv, dacc.at[dstv.at[i]], ds, add=True)
      return carry

    lax.fori_loop(0, n_chunks // 2, pairs, 0)
    pltpu.make_async_copy(onesv, dacc.at[dstv.at[0]], ds0).wait()
    pltpu.make_async_copy(onesv, dacc.at[dstv.at[0]], ds1).wait()
    plsc.subcore_barrier()
    off = 0
    while off < rpt:
      w = min(CH, rpt - off)
      pltpu.sync_copy(dacc.at[pl.ds(base + off, w)], zbuf.at[pl.ds(0, w)])
      pltpu.sync_copy(zbuf.at[pl.ds(0, w)], degp.at[c, pl.ds(base + off, w)])
      off += w

  return pl.kernel(
      body,
      out_type=[jax.ShapeDtypeStruct((NC, n_pad, 8), F32)],
      mesh=_MESH,
      compiler_params=pltpu.CompilerParams(use_tc_tiling_on_sc=False),
      scratch_types=[
          pltpu.VMEM((n_chunks, CH), jnp.int32),
          pltpu.VMEM((CH, 8), F32),
          pltpu.VMEM((CH, 8), F32),
          pltpu.VMEM_SHARED((n_pad, 8), F32),
          pltpu.SemaphoreType.DMA,
          pltpu.SemaphoreType.DMA,
      ],
  )


def _sc_agg64(n_pad, n_chunks):
  assert n_chunks % 4 == 0
  q = n_chunks // 4

  def body(table2, src, dst, zeros64, part,
           srcv, dstv, r0, r1, r2, r3, acc,
           g0, g1, g2, g3, s0, s1, s2, s3):
    c = lax.axis_index("c")
    s = lax.axis_index("s")
    rpt = n_pad // NS
    base = s * rpt
    rows = (r0, r1, r2, r3)
    gsem = (g0, g1, g2, g3)
    ssem = (s0, s1, s2, s3)
    pltpu.sync_copy(zeros64, r0)
    off = 0
    while off < rpt:
      w = min(CH, rpt - off)
      pltpu.sync_copy(r0.at[pl.ds(0, w)], acc.at[pl.ds(base + off, w)])
      off += w
    pltpu.sync_copy(src.at[s], srcv)
    pltpu.sync_copy(dst.at[s], dstv)
    plsc.subcore_barrier()

    tab = table2.at[c]
    pltpu.async_copy(tab.at[srcv.at[0]], r0, g0)
    pltpu.async_copy(tab.at[srcv.at[1]], r1, g1)

    def group(j, carry):
      for k in range(4):
        i = 4 * j + k
        kn = (k + 2) % 4
        pltpu.make_async_copy(tab.at[srcv.at[0]], rows[k], gsem[k]).wait()
        pltpu.async_copy(rows[k], acc.at[dstv.at[i]], ssem[k], add=True)
        def wait_s(kn=kn):
          pltpu.make_async_copy(rows[kn], acc.at[dstv.at[0]],
                                ssem[kn]).wait()

        def issue_g(i=i, kn=kn):
          pltpu.async_copy(tab.at[srcv.at[i + 2]], rows[kn], gsem[kn])
        if k < 2:
          pl.when(j > 0)(wait_s)
          issue_g()
        else:
          wait_s()
          pl.when(j < q - 1)(issue_g)
      return carry

    lax.fori_loop(0, q, group, 0)
    pltpu.make_async_copy(rows[(n_chunks - 2) % 4], acc.at[dstv.at[0]],
                          ssem[(n_chunks - 2) % 4]).wait()
    pltpu.make_async_copy(rows[(n_chunks - 1) % 4], acc.at[dstv.at[0]],
                          ssem[(n_chunks - 1) % 4]).wait()
    plsc.subcore_barrier()
    off = 0
    while off < rpt:
      w = min(CH, rpt - off)
      pltpu.sync_copy(acc.at[pl.ds(base + off, w)], r0.at[pl.ds(0, w)])
      pltpu.sync_copy(r0.at[pl.ds(0, w)], part.at[c, pl.ds(base + off, w)])
      off += w

  return pl.kernel(
      body,
      out_type=[jax.ShapeDtypeStruct((NC, n_pad, 64), F32)],
      mesh=_MESH,
      compiler_params=pltpu.CompilerParams(use_tc_tiling_on_sc=False),
      scratch_types=[
          pltpu.VMEM((n_chunks, CH), jnp.int32),
          pltpu.VMEM((n_chunks, CH), jnp.int32),
          pltpu.VMEM((CH, 64), F32),
          pltpu.VMEM((CH, 64), F32),
          pltpu.VMEM((CH, 64), F32),
          pltpu.VMEM((CH, 64), F32),
          pltpu.VMEM_SHARED((n_pad, 64), F32),
          pltpu.SemaphoreType.DMA,
          pltpu.SemaphoreType.DMA,
          pltpu.SemaphoreType.DMA,
          pltpu.SemaphoreType.DMA,
          pltpu.SemaphoreType.DMA,
          pltpu.SemaphoreType.DMA,
          pltpu.SemaphoreType.DMA,
          pltpu.SemaphoreType.DMA,
      ],
  )


def _sc_agg8(n_pad, n_chunks):
  assert n_chunks % 4 == 0
  q = n_chunks // 4

  def body(table, src, dst, zeros8, part,
           srcv, dstv, r0, r1, r2, r3, acc,
           g0, g1, g2, g3, s0, s1, s2, s3):
    c = lax.axis_index("c")
    s = lax.axis_index("s")
    wid = s * NC + c
    rpt = n_pad // NS
    base = s * rpt
    rows = (r0, r1, r2, r3)
    gsem = (g0, g1, g2, g3)
    ssem = (s0, s1, s2, s3)
    pltpu.sync_copy(zeros8, r0)
    off = 0
    while off < rpt:
      w = min(CH, rpt - off)
      pltpu.sync_copy(r0.at[pl.ds(0, w)], acc.at[pl.ds(base + off, w)])
      off += w
    pltpu.sync_copy(src.at[wid], srcv)
    pltpu.sync_copy(dst.at[wid], dstv)
    plsc.subcore_barrier()

    pltpu.async_copy(table.at[srcv.at[0]], r0, g0)
    pltpu.async_copy(table.at[srcv.at[1]], r1, g1)

    def group(j, carry):
      for k in range(4):
        i = 4 * j + k
        kn = (k + 2) % 4
        pltpu.make_async_copy(table.at[srcv.at[0]], rows[k], gsem[k]).wait()
        pltpu.async_copy(rows[k], acc.at[dstv.at[i]], ssem[k], add=True)
        def wait_s(kn=kn):
          pltpu.make_async_copy(rows[kn], acc.at[dstv.at[0]],
                                ssem[kn]).wait()

        def issue_g(i=i, kn=kn):
          pltpu.async_copy(table.at[srcv.at[i + 2]], rows[kn], gsem[kn])
        if k < 2:
          pl.when(j > 0)(wait_s)
          issue_g()
        else:
          wait_s()
          pl.when(j < q - 1)(issue_g)
      return carry

    lax.fori_loop(0, q, group, 0)
    pltpu.make_async_copy(rows[(n_chunks - 2) % 4], acc.at[dstv.at[0]],
                          ssem[(n_chunks - 2) % 4]).wait()
    pltpu.make_async_copy(rows[(n_chunks - 1) % 4], acc.at[dstv.at[0]],
                          ssem[(n_chunks - 1) % 4]).wait()
    plsc.subcore_barrier()
    off = 0
    while off < rpt:
      w = min(CH, rpt - off)
      pltpu.sync_copy(acc.at[pl.ds(base + off, w)], r0.at[pl.ds(0, w)])
      pltpu.sync_copy(r0.at[pl.ds(0, w)], part.at[c, pl.ds(base + off, w)])
      off += w

  return pl.kernel(
      body,
      out_type=[jax.ShapeDtypeStruct((NC, n_pad, 8), F32)],
      mesh=_MESH,
      compiler_params=pltpu.CompilerParams(use_tc_tiling_on_sc=False),
      scratch_types=[
          pltpu.VMEM((n_chunks, CH), jnp.int32),
          pltpu.VMEM((n_chunks, CH), jnp.int32),
          pltpu.VMEM((CH, 8), F32),
          pltpu.VMEM((CH, 8), F32),
          pltpu.VMEM((CH, 8), F32),
          pltpu.VMEM((CH, 8), F32),
          pltpu.VMEM_SHARED((n_pad, 8), F32),
          pltpu.SemaphoreType.DMA,
          pltpu.SemaphoreType.DMA,
          pltpu.SemaphoreType.DMA,
          pltpu.SemaphoreType.DMA,
          pltpu.SemaphoreType.DMA,
          pltpu.SemaphoreType.DMA,
          pltpu.SemaphoreType.DMA,
          pltpu.SemaphoreType.DMA,
      ],
  )


def _tc_transform(n, d_in, bm):
  def body(h, wl, wr, b, ztab, sout):
    hb = h[...]
    z = jnp.dot(hb, wl[...], preferred_element_type=F32)
    ztab[0] = z[:, :64]
    ztab[1] = z[:, 64:]
    sout[...] = jnp.dot(hb, wr[...], preferred_element_type=F32) + b[...]

  return pl.pallas_call(
      body,
      grid=(n // bm,),
      in_specs=[
          pl.BlockSpec((bm, d_in), lambda i: (i, 0)),
          pl.BlockSpec((d_in, 128), lambda i: (0, 0)),
          pl.BlockSpec((d_in, 128), lambda i: (0, 0)),
          pl.BlockSpec((1, 128), lambda i: (0, 0)),
      ],
      out_specs=[
          pl.BlockSpec((2, bm, 64), lambda i: (0, i, 0)),
          pl.BlockSpec((bm, 128), lambda i: (i, 0)),
      ],
      out_shape=[
          jax.ShapeDtypeStruct((2, n, 64), F32),
          jax.ShapeDtypeStruct((n, 128), F32),
      ],
  )


def _tc_combine_transform(n, bm, table_out, d_extra):
  def body(p, degp, s, wc, bc, *outs):
    deg = degp[0, :, 0:1] + degp[1, :, 0:1]
    invd = 1.0 / jnp.maximum(deg, 1.0)
    agg = jnp.concatenate([p[0], p[1]], axis=1)
    h = jnp.maximum(agg * invd + s[...], 0.0)
    y = jnp.dot(h, wc[...], preferred_element_type=F32) + bc[...]
    if table_out:
      outs[0][0] = y[:, :64]
      outs[0][1] = y[:, 64:128]
      outs[1][...] = y[:, 128:]
    else:
      outs[0][...] = y

  d_out = (128 if table_out else 0) + d_extra
  if table_out:
    out_specs = [pl.BlockSpec((2, bm, 64), lambda i: (0, i, 0)),
                 pl.BlockSpec((bm, d_extra), lambda i: (i, 0))]
    out_shape = [jax.ShapeDtypeStruct((2, n, 64), F32),
                 jax.ShapeDtypeStruct((n, d_extra), F32)]
  else:
    out_specs = [pl.BlockSpec((bm, d_extra), lambda i: (i, 0))]
    out_shape = [jax.ShapeDtypeStruct((n, d_extra), F32)]

  return pl.pallas_call(
      body,
      grid=(n // bm,),
      in_specs=[
          pl.BlockSpec((2, bm, 64), lambda i: (0, i, 0)),
          pl.BlockSpec((2, bm, 8), lambda i: (0, i, 0)),
          pl.BlockSpec((bm, 128), lambda i: (i, 0)),
          pl.BlockSpec((128, d_out), lambda i: (0, 0)),
          pl.BlockSpec((1, d_out), lambda i: (0, 0)),
      ],
      out_specs=out_specs,
      out_shape=out_shape,
  )


def _tc_final(n, bm):
  def body(p, degp, t3, out):
    deg = degp[0, :, 0:1] + degp[1, :, 0:1]
    invd = 1.0 / jnp.maximum(deg, 1.0)
    agg = p[0, :, 0:1] + p[1, :, 0:1]
    out[...] = agg * invd + t3[:, 1:2]

  return pl.pallas_call(
      body,
      grid=(n // bm,),
      in_specs=[
          pl.BlockSpec((2, bm, 8), lambda i: (0, i, 0)),
          pl.BlockSpec((2, bm, 8), lambda i: (0, i, 0)),
          pl.BlockSpec((bm, 8), lambda i: (i, 0)),
      ],
      out_specs=pl.BlockSpec((bm, 1), lambda i: (i, 0)),
      out_shape=jax.ShapeDtypeStruct((n, 1), F32),
  )


def _pad_edges(idx, nsplit, fill):
  per = idx.shape[0] // nsplit
  n_chunks = 4 * pl.cdiv(per, 4 * CH)
  pad = n_chunks * CH - per
  out = jnp.pad(idx.reshape(nsplit, per), ((0, 0), (0, pad)),
                constant_values=fill)
  return out.reshape(nsplit, n_chunks, CH), n_chunks


def kernel(x, edge_index, W1l, b1l, W1r, W2l, b2l, W2r, W3l, b3l, W3r):
  n, d_in = x.shape
  e = edge_index.shape[1]
  assert e % NW == 0 and e % NS == 0
  n_pad = pl.cdiv(n, 128) * 128
  bm = 2000

  src32, ch32 = _pad_edges(edge_index[0], NW, 0)
  dst32, _ = _pad_edges(edge_index[1], NW, n)
  src16, ch16 = _pad_edges(edge_index[0], NS, 0)
  dst16, _ = _pad_edges(edge_index[1], NS, n)
  zeros64 = jnp.zeros((CH, 64), F32)
  zeros8 = jnp.zeros((CH, 8), F32)
  ones_h = jnp.ones((CH, 8), F32)

  deg_k = _sc_deg(n_pad, ch32)
  agg64 = _sc_agg64(n_pad, ch16)
  agg8 = _sc_agg8(n_pad, ch32)

  (degp,) = deg_k(dst32, zeros8, ones_h)

  z1, s1 = _tc_transform(n, d_in, bm)(x, W1l.T, W1r.T, b1l[None, :])
  (p1,) = agg64(z1, src16, dst16, zeros64)

  wc2 = jnp.concatenate([W2l.T, W2r.T], axis=1)
  bc2 = jnp.concatenate([jnp.zeros((128,), F32), b2l])[None, :]
  z2, s2 = _tc_combine_transform(n, bm, True, 128)(p1, degp, s1, wc2, bc2)
  (p2,) = agg64(z2, src16, dst16, zeros64)

  wc3 = jnp.concatenate([W3l.T, W3r.T, jnp.zeros((128, 6), F32)], axis=1)
  bc3 = jnp.concatenate([jnp.zeros((1,), F32), b3l,
                         jnp.zeros((6,), F32)])[None, :]
  (t3,) = _tc_combine_transform(n, bm, False, 8)(p2, degp, s2, wc3, bc3)
  (p3,) = agg8(t3, src32, dst32, zeros8)

  return _tc_final(n, bm)(p3, degp, t3)


# speedup vs baseline: 1.2386x; 1.2064x over previous
import functools

import jax
import jax.numpy as jnp
from jax import lax
from jax.experimental import pallas as pl
from jax.experimental.pallas import tpu as pltpu
from jax.experimental.pallas import tpu_sc as plsc

F32 = jnp.float32

NC = 2
NS = 16
NW = NC * NS
CH = 128

_MESH = plsc.VectorSubcoreMesh(core_axis_name="c", subcore_axis_name="s")


def _sc_deg(n_pad, n_chunks):
  def body(dst, zeros8, ones_h, degp, dstv, onesv, zbuf, dacc, ds0, ds1):
    c = lax.axis_index("c")
    s = lax.axis_index("s")
    wid = s * NC + c
    rpt = n_pad // NS
    base = s * rpt
    pltpu.sync_copy(zeros8, zbuf)

    def init(t, carry):
      pltpu.sync_copy(zbuf, dacc.at[pl.ds(base + t * CH, CH)])
      return carry

    lax.fori_loop(0, rpt // CH, init, 0)
    pltpu.sync_copy(ones_h, onesv)
    pltpu.sync_copy(dst.at[wid], dstv)
    plsc.subcore_barrier()

    def pairs(j, carry):
      for k, ds in ((0, ds0), (1, ds1)):
        i = 2 * j + k

        @pl.when(j > 0)
        def _():
          pltpu.make_async_copy(onesv, dacc.at[dstv.at[0]], ds).wait()

        pltpu.async_copy(onesv, dacc.at[dstv.at[i]], ds, add=True)
      return carry

    lax.fori_loop(0, n_chunks // 2, pairs, 0)
    pltpu.make_async_copy(onesv, dacc.at[dstv.at[0]], ds0).wait()
    pltpu.make_async_copy(onesv, dacc.at[dstv.at[0]], ds1).wait()
    plsc.subcore_barrier()

    def rdout(t, carry):
      o = pl.ds(base + t * CH, CH)
      pltpu.sync_copy(dacc.at[o], zbuf)
      pltpu.sync_copy(zbuf, degp.at[c, o])
      return carry

    lax.fori_loop(0, rpt // CH, rdout, 0)

  return pl.kernel(
      body,
      out_type=[jax.ShapeDtypeStruct((NC, n_pad, 8), F32)],
      mesh=_MESH,
      compiler_params=pltpu.CompilerParams(use_tc_tiling_on_sc=False),
      scratch_types=[
          pltpu.VMEM((n_chunks, CH), jnp.int32),
          pltpu.VMEM((CH, 8), F32),
          pltpu.VMEM((CH, 8), F32),
          pltpu.VMEM_SHARED((n_pad, 8), F32),
          pltpu.SemaphoreType.DMA,
          pltpu.SemaphoreType.DMA,
      ],
  )


def _sc_agg64(n_pad, n_chunks):
  def body(table2, src, dst, zeros64, part,
           srcv, dstv, rows0, tabs, acc, gs):
    c = lax.axis_index("c")
    s = lax.axis_index("s")
    rpt = n_pad // NS
    base = s * rpt
    pltpu.sync_copy(zeros64, rows0)

    def zinit(t, carry):
      pltpu.sync_copy(rows0, acc.at[pl.ds(base + t * CH, CH)])
      return carry

    lax.fori_loop(0, rpt // CH, zinit, 0)
    tab_hbm = table2.at[c]

    def stage(t, carry):
      o = pl.ds(base + t * CH, CH)
      pltpu.sync_copy(tab_hbm.at[o], rows0)
      pltpu.sync_copy(rows0, tabs.at[o])
      return carry

    lax.fori_loop(0, rpt // CH, stage, 0)
    pltpu.sync_copy(src.at[s], srcv)
    pltpu.sync_copy(dst.at[s], dstv)
    plsc.subcore_barrier()

    def chunk(i, carry):
      pltpu.async_copy(tabs.at[srcv.at[i]], rows0, gs).wait()
      pltpu.sync_copy(rows0, acc.at[dstv.at[i]], add=True)
      return carry

    lax.fori_loop(0, n_chunks, chunk, 0)
    plsc.subcore_barrier()

    def rdout(t, carry):
      o = pl.ds(base + t * CH, CH)
      pltpu.sync_copy(acc.at[o], rows0)
      pltpu.sync_copy(rows0, part.at[c, o])
      return carry

    lax.fori_loop(0, rpt // CH, rdout, 0)

  return pl.kernel(
      body,
      out_type=[jax.ShapeDtypeStruct((NC, n_pad, 64), F32)],
      mesh=_MESH,
      compiler_params=pltpu.CompilerParams(use_tc_tiling_on_sc=False),
      scratch_types=[
          pltpu.VMEM((n_chunks, CH), jnp.int32),
          pltpu.VMEM((n_chunks, CH), jnp.int32),
          pltpu.VMEM((CH, 64), F32),
          pltpu.VMEM_SHARED((n_pad, 64), F32),
          pltpu.VMEM_SHARED((n_pad, 64), F32),
          pltpu.SemaphoreType.DMA,
      ],
  )


def _sc_agg8(n_pad, n_chunks):
  def body(table, src, dst, zeros8, part,
           srcv, dstv, rows0, tabs, acc, gs):
    c = lax.axis_index("c")
    s = lax.axis_index("s")
    wid = s * NC + c
    rpt = n_pad // NS
    base = s * rpt
    pltpu.sync_copy(zeros8, rows0)

    def zinit(t, carry):
      pltpu.sync_copy(rows0, acc.at[pl.ds(base + t * CH, CH)])
      return carry

    lax.fori_loop(0, rpt // CH, zinit, 0)

    def stage(t, carry):
      o = pl.ds(base + t * CH, CH)
      pltpu.sync_copy(table.at[o], rows0)
      pltpu.sync_copy(rows0, tabs.at[o])
      return carry

    lax.fori_loop(0, rpt // CH, stage, 0)
    pltpu.sync_copy(src.at[wid], srcv)
    pltpu.sync_copy(dst.at[wid], dstv)
    plsc.subcore_barrier()

    def chunk(i, carry):
      pltpu.async_copy(tabs.at[srcv.at[i]], rows0, gs).wait()
      pltpu.sync_copy(rows0, acc.at[dstv.at[i]], add=True)
      return carry

    lax.fori_loop(0, n_chunks, chunk, 0)
    plsc.subcore_barrier()

    def rdout(t, carry):
      o = pl.ds(base + t * CH, CH)
      pltpu.sync_copy(acc.at[o], rows0)
      pltpu.sync_copy(rows0, part.at[c, o])
      return carry

    lax.fori_loop(0, rpt // CH, rdout, 0)

  return pl.kernel(
      body,
      out_type=[jax.ShapeDtypeStruct((NC, n_pad, 8), F32)],
      mesh=_MESH,
      compiler_params=pltpu.CompilerParams(use_tc_tiling_on_sc=False),
      scratch_types=[
          pltpu.VMEM((n_chunks, CH), jnp.int32),
          pltpu.VMEM((n_chunks, CH), jnp.int32),
          pltpu.VMEM((CH, 8), F32),
          pltpu.VMEM_SHARED((n_pad, 8), F32),
          pltpu.VMEM_SHARED((n_pad, 8), F32),
          pltpu.SemaphoreType.DMA,
      ],
  )


def _tc_transform(n, n_pad, d_in, bm):
  def body(h, wl, wr, b, ztab, sout):
    hb = h[...]
    z = jnp.dot(hb, wl[...], preferred_element_type=F32)
    ztab[0] = z[:, :64]
    ztab[1] = z[:, 64:]
    sout[...] = jnp.dot(hb, wr[...], preferred_element_type=F32) + b[...]

  return pl.pallas_call(
      body,
      grid=(n // bm,),
      in_specs=[
          pl.BlockSpec((bm, d_in), lambda i: (i, 0)),
          pl.BlockSpec((d_in, 128), lambda i: (0, 0)),
          pl.BlockSpec((d_in, 128), lambda i: (0, 0)),
          pl.BlockSpec((1, 128), lambda i: (0, 0)),
      ],
      out_specs=[
          pl.BlockSpec((2, bm, 64), lambda i: (0, i, 0)),
          pl.BlockSpec((bm, 128), lambda i: (i, 0)),
      ],
      out_shape=[
          jax.ShapeDtypeStruct((2, n_pad, 64), F32),
          jax.ShapeDtypeStruct((n, 128), F32),
      ],
  )


def _tc_combine_transform(n, n_pad, bm, table_out, d_extra):
  def body(p, degp, s, wc, bc, *outs):
    deg = degp[0, :, 0:1] + degp[1, :, 0:1]
    invd = 1.0 / jnp.maximum(deg, 1.0)
    agg = jnp.concatenate([p[0], p[1]], axis=1)
    h = jnp.maximum(agg * invd + s[...], 0.0)
    y = jnp.dot(h, wc[...], preferred_element_type=F32) + bc[...]
    if table_out:
      outs[0][0] = y[:, :64]
      outs[0][1] = y[:, 64:128]
      outs[1][...] = y[:, 128:]
    else:
      outs[0][...] = y

  d_out = (128 if table_out else 0) + d_extra
  if table_out:
    out_specs = [pl.BlockSpec((2, bm, 64), lambda i: (0, i, 0)),
                 pl.BlockSpec((bm, d_extra), lambda i: (i, 0))]
    out_shape = [jax.ShapeDtypeStruct((2, n_pad, 64), F32),
                 jax.ShapeDtypeStruct((n, d_extra), F32)]
  else:
    out_specs = [pl.BlockSpec((bm, d_extra), lambda i: (i, 0))]
    out_shape = [jax.ShapeDtypeStruct((n_pad, d_extra), F32)]

  return pl.pallas_call(
      body,
      grid=(n // bm,),
      in_specs=[
          pl.BlockSpec((2, bm, 64), lambda i: (0, i, 0)),
          pl.BlockSpec((2, bm, 8), lambda i: (0, i, 0)),
          pl.BlockSpec((bm, 128), lambda i: (i, 0)),
          pl.BlockSpec((128, d_out), lambda i: (0, 0)),
          pl.BlockSpec((1, d_out), lambda i: (0, 0)),
      ],
      out_specs=out_specs,
      out_shape=out_shape,
  )


def _tc_final(n, bm):
  def body(p, degp, t3, out):
    deg = degp[0, :, 0:1] + degp[1, :, 0:1]
    invd = 1.0 / jnp.maximum(deg, 1.0)
    agg = p[0, :, 0:1] + p[1, :, 0:1]
    out[...] = agg * invd + t3[:, 1:2]

  return pl.pallas_call(
      body,
      grid=(n // bm,),
      in_specs=[
          pl.BlockSpec((2, bm, 8), lambda i: (0, i, 0)),
          pl.BlockSpec((2, bm, 8), lambda i: (0, i, 0)),
          pl.BlockSpec((bm, 8), lambda i: (i, 0)),
      ],
      out_specs=pl.BlockSpec((bm, 1), lambda i: (i, 0)),
      out_shape=jax.ShapeDtypeStruct((n, 1), F32),
  )


def _pad_edges(idx, nsplit, fill):
  per = idx.shape[0] // nsplit
  n_chunks = 4 * pl.cdiv(per, 4 * CH)
  pad = n_chunks * CH - per
  out = jnp.pad(idx.reshape(nsplit, per), ((0, 0), (0, pad)),
                constant_values=fill)
  return out.reshape(nsplit, n_chunks, CH), n_chunks


def kernel(x, edge_index, W1l, b1l, W1r, W2l, b2l, W2r, W3l, b3l, W3r):
  n, d_in = x.shape
  e = edge_index.shape[1]
  assert e % NW == 0 and e % NS == 0
  n_pad = pl.cdiv(n, NS * CH) * NS * CH
  bm = 2000

  src32, ch32 = _pad_edges(edge_index[0], NW, 0)
  dst32, _ = _pad_edges(edge_index[1], NW, n)
  src16, ch16 = _pad_edges(edge_index[0], NS, 0)
  dst16, _ = _pad_edges(edge_index[1], NS, n)
  zeros64 = jnp.zeros((CH, 64), F32)
  zeros8 = jnp.zeros((CH, 8), F32)
  ones_h = jnp.ones((CH, 8), F32)

  deg_k = _sc_deg(n_pad, ch32)
  agg64 = _sc_agg64(n_pad, ch16)
  agg8 = _sc_agg8(n_pad, ch32)

  (degp,) = deg_k(dst32, zeros8, ones_h)

  z1, s1 = _tc_transform(n, n_pad, d_in, bm)(x, W1l.T, W1r.T, b1l[None, :])
  (p1,) = agg64(z1, src16, dst16, zeros64)

  wc2 = jnp.concatenate([W2l.T, W2r.T], axis=1)
  bc2 = jnp.concatenate([jnp.zeros((128,), F32), b2l])[None, :]
  z2, s2 = _tc_combine_transform(n, n_pad, bm, True, 128)(p1, degp, s1, wc2, bc2)
  (p2,) = agg64(z2, src16, dst16, zeros64)

  wc3 = jnp.concatenate([W3l.T, W3r.T, jnp.zeros((128, 6), F32)], axis=1)
  bc3 = jnp.concatenate([jnp.zeros((1,), F32), b3l,
                         jnp.zeros((6,), F32)])[None, :]
  (t3,) = _tc_combine_transform(n, n_pad, bm, False, 8)(p2, degp, s2, wc3, bc3)
  (p3,) = agg8(t3, src32, dst32, zeros8)

  return _tc_final(n, bm)(p3, degp, t3)


# speedup vs baseline: 1.5756x; 1.2721x over previous
import functools

import jax
import jax.numpy as jnp
from jax import lax
from jax.experimental import pallas as pl
from jax.experimental.pallas import tpu as pltpu
from jax.experimental.pallas import tpu_sc as plsc

F32 = jnp.float32

NC = 2
NS = 16
NW = NC * NS
CH = 64

_MESH = plsc.VectorSubcoreMesh(core_axis_name="c", subcore_axis_name="s")


def _sc_deg(n_pad, n_chunks):
  def body(dst, zeros8, ones_h, degp, dstv, onesv, zbuf, dacc, ds0, ds1):
    c = lax.axis_index("c")
    s = lax.axis_index("s")
    wid = s * NC + c
    rpt = n_pad // NS
    base = s * rpt
    pltpu.sync_copy(zeros8, zbuf)

    def init(t, carry):
      pltpu.sync_copy(zbuf, dacc.at[pl.ds(base + t * CH, CH)])
      return carry

    lax.fori_loop(0, rpt // CH, init, 0)
    pltpu.sync_copy(ones_h, onesv)
    pltpu.sync_copy(dst.at[wid], dstv)
    plsc.subcore_barrier()

    def pairs(j, carry):
      for k, ds in ((0, ds0), (1, ds1)):
        i = 2 * j + k

        @pl.when(j > 0)
        def _():
          pltpu.make_async_copy(onesv, dacc.at[dstv.at[0]], ds).wait()

        pltpu.async_copy(onesv, dacc.at[dstv.at[i]], ds, add=True)
      return carry

    lax.fori_loop(0, n_chunks // 2, pairs, 0)
    pltpu.make_async_copy(onesv, dacc.at[dstv.at[0]], ds0).wait()
    pltpu.make_async_copy(onesv, dacc.at[dstv.at[0]], ds1).wait()
    plsc.subcore_barrier()

    def rdout(t, carry):
      o = pl.ds(base + t * CH, CH)
      pltpu.sync_copy(dacc.at[o], zbuf)
      pltpu.sync_copy(zbuf, degp.at[c, o])
      return carry

    lax.fori_loop(0, rpt // CH, rdout, 0)

  return pl.kernel(
      body,
      out_type=[jax.ShapeDtypeStruct((NC, n_pad, 8), F32)],
      mesh=_MESH,
      compiler_params=pltpu.CompilerParams(use_tc_tiling_on_sc=False),
      scratch_types=[
          pltpu.VMEM((n_chunks, CH), jnp.int32),
          pltpu.VMEM((CH, 8), F32),
          pltpu.VMEM((CH, 8), F32),
          pltpu.VMEM_SHARED((n_pad, 8), F32),
          pltpu.SemaphoreType.DMA,
          pltpu.SemaphoreType.DMA,
      ],
  )


def _sc_agg64(n_pad, n_chunks):
  def body(table2, src, dst, zeros64, part,
           srcv, dstv, rows0, rows1, tabs, acc, gs0, gs1, ss0, ss1):
    c = lax.axis_index("c")
    s = lax.axis_index("s")
    rpt = n_pad // NS
    base = s * rpt
    pltpu.sync_copy(zeros64, rows0)

    def zinit(t, carry):
      pltpu.sync_copy(rows0, acc.at[pl.ds(base + t * CH, CH)])
      return carry

    lax.fori_loop(0, rpt // CH, zinit, 0)
    tab_hbm = table2.at[c]

    def stage(t, carry):
      o = pl.ds(base + t * CH, CH)
      pltpu.sync_copy(tab_hbm.at[o], rows0)
      pltpu.sync_copy(rows0, tabs.at[o])
      return carry

    lax.fori_loop(0, rpt // CH, stage, 0)
    pltpu.sync_copy(src.at[s], srcv)
    pltpu.sync_copy(dst.at[s], dstv)
    plsc.subcore_barrier()

    def pair(j, carry):
      for k, (rows, gs, ss) in ((0, (rows0, gs0, ss0)),
                                (1, (rows1, gs1, ss1))):
        i = 2 * j + k

        @pl.when(j > 0)
        def _():
          pltpu.make_async_copy(rows, acc.at[dstv.at[0]], ss).wait()

        pltpu.async_copy(tabs.at[srcv.at[i]], rows, gs).wait()
        pltpu.async_copy(rows, acc.at[dstv.at[i]], ss, add=True)
      return carry

    lax.fori_loop(0, n_chunks // 2, pair, 0)
    pltpu.make_async_copy(rows0, acc.at[dstv.at[0]], ss0).wait()
    pltpu.make_async_copy(rows1, acc.at[dstv.at[0]], ss1).wait()
    plsc.subcore_barrier()

    def rdout(t, carry):
      o = pl.ds(base + t * CH, CH)
      pltpu.sync_copy(acc.at[o], rows0)
      pltpu.sync_copy(rows0, part.at[c, o])
      return carry

    lax.fori_loop(0, rpt // CH, rdout, 0)

  return pl.kernel(
      body,
      out_type=[jax.ShapeDtypeStruct((NC, n_pad, 64), F32)],
      mesh=_MESH,
      compiler_params=pltpu.CompilerParams(use_tc_tiling_on_sc=False),
      scratch_types=[
          pltpu.VMEM((n_chunks, CH), jnp.int32),
          pltpu.VMEM((n_chunks, CH), jnp.int32),
          pltpu.VMEM((CH, 64), F32),
          pltpu.VMEM((CH, 64), F32),
          pltpu.VMEM_SHARED((n_pad, 64), F32),
          pltpu.VMEM_SHARED((n_pad, 64), F32),
          pltpu.SemaphoreType.DMA,
          pltpu.SemaphoreType.DMA,
          pltpu.SemaphoreType.DMA,
          pltpu.SemaphoreType.DMA,
      ],
  )


def _sc_agg8(n_pad, n_chunks):
  def body(table, src, dst, zeros8, part,
           srcv, dstv, rows0, rows1, tabs, acc, gs0, gs1, ss0, ss1):
    c = lax.axis_index("c")
    s = lax.axis_index("s")
    wid = s * NC + c
    rpt = n_pad // NS
    base = s * rpt
    pltpu.sync_copy(zeros8, rows0)

    def zinit(t, carry):
      pltpu.sync_copy(rows0, acc.at[pl.ds(base + t * CH, CH)])
      return carry

    lax.fori_loop(0, rpt // CH, zinit, 0)

    def stage(t, carry):
      o = pl.ds(base + t * CH, CH)
      pltpu.sync_copy(table.at[o], rows0)
      pltpu.sync_copy(rows0, tabs.at[o])
      return carry

    lax.fori_loop(0, rpt // CH, stage, 0)
    pltpu.sync_copy(src.at[wid], srcv)
    pltpu.sync_copy(dst.at[wid], dstv)
    plsc.subcore_barrier()

    def pair(j, carry):
      for k, (rows, gs, ss) in ((0, (rows0, gs0, ss0)),
                                (1, (rows1, gs1, ss1))):
        i = 2 * j + k

        @pl.when(j > 0)
        def _():
          pltpu.make_async_copy(rows, acc.at[dstv.at[0]], ss).wait()

        pltpu.async_copy(tabs.at[srcv.at[i]], rows, gs).wait()
        pltpu.async_copy(rows, acc.at[dstv.at[i]], ss, add=True)
      return carry

    lax.fori_loop(0, n_chunks // 2, pair, 0)
    pltpu.make_async_copy(rows0, acc.at[dstv.at[0]], ss0).wait()
    pltpu.make_async_copy(rows1, acc.at[dstv.at[0]], ss1).wait()
    plsc.subcore_barrier()

    def rdout(t, carry):
      o = pl.ds(base + t * CH, CH)
      pltpu.sync_copy(acc.at[o], rows0)
      pltpu.sync_copy(rows0, part.at[c, o])
      return carry

    lax.fori_loop(0, rpt // CH, rdout, 0)

  return pl.kernel(
      body,
      out_type=[jax.ShapeDtypeStruct((NC, n_pad, 8), F32)],
      mesh=_MESH,
      compiler_params=pltpu.CompilerParams(use_tc_tiling_on_sc=False),
      scratch_types=[
          pltpu.VMEM((n_chunks, CH), jnp.int32),
          pltpu.VMEM((n_chunks, CH), jnp.int32),
          pltpu.VMEM((CH, 8), F32),
          pltpu.VMEM((CH, 8), F32),
          pltpu.VMEM_SHARED((n_pad, 8), F32),
          pltpu.VMEM_SHARED((n_pad, 8), F32),
          pltpu.SemaphoreType.DMA,
          pltpu.SemaphoreType.DMA,
          pltpu.SemaphoreType.DMA,
          pltpu.SemaphoreType.DMA,
      ],
  )


def _tc_transform(n, n_pad, d_in, bm):
  def body(h, wl, wr, b, ztab, sout):
    hb = h[...]
    z = jnp.dot(hb, wl[...], preferred_element_type=F32)
    ztab[0] = z[:, :64]
    ztab[1] = z[:, 64:]
    sout[...] = jnp.dot(hb, wr[...], preferred_element_type=F32) + b[...]

  return pl.pallas_call(
      body,
      grid=(n // bm,),
      in_specs=[
          pl.BlockSpec((bm, d_in), lambda i: (i, 0)),
          pl.BlockSpec((d_in, 128), lambda i: (0, 0)),
          pl.BlockSpec((d_in, 128), lambda i: (0, 0)),
          pl.BlockSpec((1, 128), lambda i: (0, 0)),
      ],
      out_specs=[
          pl.BlockSpec((2, bm, 64), lambda i: (0, i, 0)),
          pl.BlockSpec((bm, 128), lambda i: (i, 0)),
      ],
      out_shape=[
          jax.ShapeDtypeStruct((2, n_pad, 64), F32),
          jax.ShapeDtypeStruct((n, 128), F32),
      ],
  )


def _tc_combine_transform(n, n_pad, bm, table_out, d_extra):
  def body(p, degp, s, wc, bc, *outs):
    deg = degp[0, :, 0:1] + degp[1, :, 0:1]
    invd = 1.0 / jnp.maximum(deg, 1.0)
    agg = jnp.concatenate([p[0], p[1]], axis=1)
    h = jnp.maximum(agg * invd + s[...], 0.0)
    y = jnp.dot(h, wc[...], preferred_element_type=F32) + bc[...]
    if table_out:
      outs[0][0] = y[:, :64]
      outs[0][1] = y[:, 64:128]
      outs[1][...] = y[:, 128:]
    else:
      outs[0][...] = y

  d_out = (128 if table_out else 0) + d_extra
  if table_out:
    out_specs = [pl.BlockSpec((2, bm, 64), lambda i: (0, i, 0)),
                 pl.BlockSpec((bm, d_extra), lambda i: (i, 0))]
    out_shape = [jax.ShapeDtypeStruct((2, n_pad, 64), F32),
                 jax.ShapeDtypeStruct((n, d_extra), F32)]
  else:
    out_specs = [pl.BlockSpec((bm, d_extra), lambda i: (i, 0))]
    out_shape = [jax.ShapeDtypeStruct((n_pad, d_extra), F32)]

  return pl.pallas_call(
      body,
      grid=(n // bm,),
      in_specs=[
          pl.BlockSpec((2, bm, 64), lambda i: (0, i, 0)),
          pl.BlockSpec((2, bm, 8), lambda i: (0, i, 0)),
          pl.BlockSpec((bm, 128), lambda i: (i, 0)),
          pl.BlockSpec((128, d_out), lambda i: (0, 0)),
          pl.BlockSpec((1, d_out), lambda i: (0, 0)),
      ],
      out_specs=out_specs,
      out_shape=out_shape,
  )


def _tc_final(n, bm):
  def body(p, degp, t3, out):
    deg = degp[0, :, 0:1] + degp[1, :, 0:1]
    invd = 1.0 / jnp.maximum(deg, 1.0)
    agg = p[0, :, 0:1] + p[1, :, 0:1]
    out[...] = agg * invd + t3[:, 1:2]

  return pl.pallas_call(
      body,
      grid=(n // bm,),
      in_specs=[
          pl.BlockSpec((2, bm, 8), lambda i: (0, i, 0)),
          pl.BlockSpec((2, bm, 8), lambda i: (0, i, 0)),
          pl.BlockSpec((bm, 8), lambda i: (i, 0)),
      ],
      out_specs=pl.BlockSpec((bm, 1), lambda i: (i, 0)),
      out_shape=jax.ShapeDtypeStruct((n, 1), F32),
  )


def _pad_edges(idx, nsplit, fill):
  per = idx.shape[0] // nsplit
  n_chunks = 2 * pl.cdiv(per, 2 * CH)
  pad = n_chunks * CH - per
  out = jnp.pad(idx.reshape(nsplit, per), ((0, 0), (0, pad)),
                constant_values=fill)
  return out.reshape(nsplit, n_chunks, CH), n_chunks


def kernel(x, edge_index, W1l, b1l, W1r, W2l, b2l, W2r, W3l, b3l, W3r):
  n, d_in = x.shape
  e = edge_index.shape[1]
  assert e % NW == 0 and e % NS == 0
  n_pad = pl.cdiv(n, NS * CH) * NS * CH
  bm = 2000

  src32, ch32 = _pad_edges(edge_index[0], NW, 0)
  dst32, _ = _pad_edges(edge_index[1], NW, n)
  src16, ch16 = _pad_edges(edge_index[0], NS, 0)
  dst16, _ = _pad_edges(edge_index[1], NS, n)
  zeros64 = jnp.zeros((CH, 64), F32)
  zeros8 = jnp.zeros((CH, 8), F32)
  ones_h = jnp.ones((CH, 8), F32)

  deg_k = _sc_deg(n_pad, ch32)
  agg64 = _sc_agg64(n_pad, ch16)
  agg8 = _sc_agg8(n_pad, ch32)

  (degp,) = deg_k(dst32, zeros8, ones_h)

  z1, s1 = _tc_transform(n, n_pad, d_in, bm)(x, W1l.T, W1r.T, b1l[None, :])
  (p1,) = agg64(z1, src16, dst16, zeros64)

  wc2 = jnp.concatenate([W2l.T, W2r.T], axis=1)
  bc2 = jnp.concatenate([jnp.zeros((128,), F32), b2l])[None, :]
  z2, s2 = _tc_combine_transform(n, n_pad, bm, True, 128)(p1, degp, s1, wc2, bc2)
  (p2,) = agg64(z2, src16, dst16, zeros64)

  wc3 = jnp.concatenate([W3l.T, W3r.T, jnp.zeros((128, 6), F32)], axis=1)
  bc3 = jnp.concatenate([jnp.zeros((1,), F32), b3l,
                         jnp.zeros((6,), F32)])[None, :]
  (t3,) = _tc_combine_transform(n, n_pad, bm, False, 8)(p2, degp, s2, wc3, bc3)
  (p3,) = agg8(t3, src32, dst32, zeros8)

  return _tc_final(n, bm)(p3, degp, t3)
